# baseline (speedup 1.0000x reference)
import functools

import jax
import jax.numpy as jnp
from jax import lax
from jax.experimental import pallas as pl
from jax.experimental.pallas import tpu as pltpu
from jax.experimental.pallas import tpu_sc as plsc

N = 1048576
HIDDEN = 64
THRESHOLD = 0.045
TAU = 20.0

R = 128
BLK = R * 128
GRID = N // BLK

NW = 32
CHUNK = N // NW
CH = 16384


def _main_body(xf_ref, w1_ref, b1_ref, w2_ref, b2_ref, w3_ref, b3_ref,
               yr_ref, nact_ref, carry_ref):
    i = pl.program_id(0)

    @pl.when(i == 0)
    def _init():
        carry_ref[0] = 0.0
        carry_ref[1] = 0.0
        carry_ref[2] = 0.0

    carry_c = carry_ref[0]
    prev_x = carry_ref[1]
    carry_v = carry_ref[2]

    xf = xf_ref[...]
    h1 = jax.nn.relu(xf * w1_ref[...] + b1_ref[...])
    h2 = jax.nn.relu(
        jnp.dot(h1, w2_ref[...], preferred_element_type=jnp.float32)
        + b2_ref[...])
    outf = (
        jnp.dot(h2, w3_ref[...], preferred_element_type=jnp.float32)
        + b3_ref[...])
    outr = outf.reshape(R, 128)

    xb = xf.reshape(R, 128)
    lane = lax.broadcasted_iota(jnp.int32, (R, 128), 1)
    rowi = lax.broadcasted_iota(jnp.int32, (R, 128), 0)
    row1 = lax.broadcasted_iota(jnp.int32, (R, 1), 0)

    xp = pltpu.roll(xb, 1, axis=1)
    col0 = pltpu.roll(xb[:, 127:128], 1, axis=0)
    col0 = jnp.where(row1 == 0, prev_x, col0)
    xp = jnp.where(lane == 0, col0, xp)

    t_f = (jnp.float32(BLK) * i.astype(jnp.float32)
           + (rowi * 128 + lane).astype(jnp.float32))
    m = jnp.abs(xb - xp) > THRESHOLD
    m = jnp.logical_or(m, t_f == 0.0)
    mf = m.astype(jnp.float32)

    ca = lax.broadcasted_iota(jnp.int32, (128, 128), 0)
    cb_ = lax.broadcasted_iota(jnp.int32, (128, 128), 1)
    tri_u = (ca <= cb_).astype(jnp.float32)
    cs = jnp.dot(mf, tri_u, preferred_element_type=jnp.float32)
    rowsum = cs[:, 127:128]
    ra = lax.broadcasted_iota(jnp.int32, (R, R), 0)
    rb = lax.broadcasted_iota(jnp.int32, (R, R), 1)
    tri_l = (rb < ra).astype(jnp.float32)
    ex = jnp.dot(tri_l, rowsum, preferred_element_type=jnp.float32)
    c_blk = cs + ex + carry_c
    carry_c_new = carry_c + jnp.sum(mf)

    dec = jnp.exp((c_blk - 1.0 - t_f) * (1.0 / TAU))

    fv = jnp.where(m, outr, 0.0)
    ff = mf
    for s in (1, 2, 4, 8, 16, 32, 64):
        sv = pltpu.roll(fv, s, axis=1)
        sf = pltpu.roll(ff, s, axis=1)
        keep = lane >= s
        sv = jnp.where(keep, sv, 0.0)
        sf = jnp.where(keep, sf, 0.0)
        fv = jnp.where(ff > 0.0, fv, sv)
        ff = jnp.maximum(ff, sf)
    rv = fv[:, 127:128]
    rf = ff[:, 127:128]
    s = 1
    row_shifts = []
    while s < R:
        row_shifts.append(s)
        s *= 2
    for s in row_shifts:
        sv = pltpu.roll(rv, s, axis=0)
        sf = pltpu.roll(rf, s, axis=0)
        keep = row1 >= s
        sv = jnp.where(keep, sv, 0.0)
        sf = jnp.where(keep, sf, 0.0)
        rv = jnp.where(rf > 0.0, rv, sv)
        rf = jnp.maximum(rf, sf)
    ev = pltpu.roll(rv, 1, axis=0)
    ef = pltpu.roll(rf, 1, axis=0)
    ev = jnp.where(jnp.logical_and(row1 >= 1, ef > 0.0), ev, carry_v)
    fill = jnp.where(ff > 0.0, fv, ev)

    yr_ref[...] = fill * dec

    carry_ref[0] = carry_c_new
    carry_ref[1] = jnp.sum(xb[R - 1:R, 127:128])
    carry_ref[2] = jnp.sum(fill[R - 1:R, 127:128])
    nact_ref[0, 0] = carry_c_new.astype(jnp.int32)


def _main_stage(x, w1r, b1r, w2t, b2r, w3t, b3r):
    return pl.pallas_call(
        _main_body,
        grid=(GRID,),
        in_specs=[
            pl.BlockSpec((BLK, 1), lambda i: (i, 0)),
            pl.BlockSpec((1, HIDDEN), lambda i: (0, 0)),
            pl.BlockSpec((1, HIDDEN), lambda i: (0, 0)),
            pl.BlockSpec((HIDDEN, HIDDEN), lambda i: (0, 0)),
            pl.BlockSpec((1, HIDDEN), lambda i: (0, 0)),
            pl.BlockSpec((HIDDEN, 1), lambda i: (0, 0)),
            pl.BlockSpec((1, 1), lambda i: (0, 0)),
        ],
        out_specs=[
            pl.BlockSpec((R, 128), lambda i: (i, 0)),
            pl.BlockSpec((1, 1), lambda i: (0, 0), memory_space=pltpu.SMEM),
        ],
        out_shape=[
            jax.ShapeDtypeStruct((N // 128, 128), jnp.float32),
            jax.ShapeDtypeStruct((1, 1), jnp.int32),
        ],
        scratch_shapes=[pltpu.SMEM((3,), jnp.float32)],
    )(x, w1r, b1r, w2t, b2r, w3t, b3r)


WB = 16384
WR = WB // 128


def _writer_body(yr_ref, o_ref):
    y = yr_ref[...]
    ii = lax.broadcasted_iota(jnp.int32, (512, 512), 0)
    jj = lax.broadcasted_iota(jnp.int32, (512, 512), 1)
    ident = (ii == jj).astype(jnp.float32)
    yflat = y.reshape(1, WB)
    cols = []
    for k in range(WB // 512):
        yc = yflat[:, k * 512:(k + 1) * 512]
        cols.append(lax.dot_general(
            ident, yc, (((1,), (1,)), ((), ())),
            preferred_element_type=jnp.float32))
    o_ref[...] = jnp.concatenate(cols, axis=0)


def _writer_stage(y_rows):
    return pl.pallas_call(
        _writer_body,
        grid=(N // WB,),
        in_specs=[pl.BlockSpec((WR, 128), lambda i: (i, 0))],
        out_specs=pl.BlockSpec((WB, 1), lambda i: (i, 0)),
        out_shape=jax.ShapeDtypeStruct((N, 1), jnp.float32),
    )(y_rows)


def kernel(x, W1, b1, W2, b2, W3, b3):
    w1r = W1.reshape(1, HIDDEN)
    b1r = b1.reshape(1, HIDDEN)
    w2t = W2.T
    b2r = b2.reshape(1, HIDDEN)
    w3t = W3.reshape(1, HIDDEN).T
    b3r = b3.reshape(1, 1)

    y_rows, nact = _main_stage(x, w1r, b1r, w2t, b2r, w3t, b3r)
    y = _writer_stage(y_rows)
    return y, nact[0, 0]


# speedup vs baseline: 1.0717x; 1.0717x over previous
import functools

import jax
import jax.numpy as jnp
from jax import lax
from jax.experimental import pallas as pl
from jax.experimental.pallas import tpu as pltpu
from jax.experimental.pallas import tpu_sc as plsc

N = 1048576
HIDDEN = 64
THRESHOLD = 0.045
TAU = 20.0

R = 128
BLK = R * 128
GRID = N // BLK

NW = 32
CHUNK = N // NW
CH = 16384


def _main_body(xf_ref, w1_ref, b1_ref, w2_ref, b2_ref, w3_ref, b3_ref,
               yr_ref, nact_ref, carry_ref):
    i = pl.program_id(0)

    @pl.when(i == 0)
    def _init():
        carry_ref[0] = 0.0
        carry_ref[1] = 0.0
        carry_ref[2] = 0.0

    carry_c = carry_ref[0]
    prev_x = carry_ref[1]
    carry_v = carry_ref[2]

    xf = xf_ref[...]
    h1 = jax.nn.relu(xf * w1_ref[...] + b1_ref[...])
    h2 = jax.nn.relu(
        jnp.dot(h1, w2_ref[...], preferred_element_type=jnp.float32)
        + b2_ref[...])
    outf = (
        jnp.dot(h2, w3_ref[...], preferred_element_type=jnp.float32)
        + b3_ref[...])
    outr = outf.reshape(R, 128)

    xb = xf.reshape(R, 128)
    lane = lax.broadcasted_iota(jnp.int32, (R, 128), 1)
    rowi = lax.broadcasted_iota(jnp.int32, (R, 128), 0)
    row1 = lax.broadcasted_iota(jnp.int32, (R, 1), 0)

    xp = pltpu.roll(xb, 1, axis=1)
    col0 = pltpu.roll(xb[:, 127:128], 1, axis=0)
    col0 = jnp.where(row1 == 0, prev_x, col0)
    xp = jnp.where(lane == 0, col0, xp)

    t_f = (jnp.float32(BLK) * i.astype(jnp.float32)
           + (rowi * 128 + lane).astype(jnp.float32))
    m = jnp.abs(xb - xp) > THRESHOLD
    m = jnp.logical_or(m, t_f == 0.0)
    mf = m.astype(jnp.float32)

    ca = lax.broadcasted_iota(jnp.int32, (128, 128), 0)
    cb_ = lax.broadcasted_iota(jnp.int32, (128, 128), 1)
    tri_u = (ca <= cb_).astype(jnp.float32)
    cs = jnp.dot(mf, tri_u, preferred_element_type=jnp.float32)
    rowsum = cs[:, 127:128]
    ra = lax.broadcasted_iota(jnp.int32, (R, R), 0)
    rb = lax.broadcasted_iota(jnp.int32, (R, R), 1)
    tri_l = (rb < ra).astype(jnp.float32)
    ex = jnp.dot(tri_l, rowsum, preferred_element_type=jnp.float32)
    c_blk = cs + ex + carry_c
    carry_c_new = carry_c + jnp.sum(mf)

    dec = jnp.exp((c_blk - 1.0 - t_f) * (1.0 / TAU))

    fv = jnp.where(m, outr, 0.0)
    ff = mf
    for s in (1, 2, 4, 8, 16, 32, 64):
        sv = pltpu.roll(fv, s, axis=1)
        sf = pltpu.roll(ff, s, axis=1)
        keep = lane >= s
        sv = jnp.where(keep, sv, 0.0)
        sf = jnp.where(keep, sf, 0.0)
        fv = jnp.where(ff > 0.0, fv, sv)
        ff = jnp.maximum(ff, sf)
    rv = fv[:, 127:128]
    rf = ff[:, 127:128]
    s = 1
    row_shifts = []
    while s < R:
        row_shifts.append(s)
        s *= 2
    for s in row_shifts:
        sv = pltpu.roll(rv, s, axis=0)
        sf = pltpu.roll(rf, s, axis=0)
        keep = row1 >= s
        sv = jnp.where(keep, sv, 0.0)
        sf = jnp.where(keep, sf, 0.0)
        rv = jnp.where(rf > 0.0, rv, sv)
        rf = jnp.maximum(rf, sf)
    ev = pltpu.roll(rv, 1, axis=0)
    ef = pltpu.roll(rf, 1, axis=0)
    ev = jnp.where(jnp.logical_and(row1 >= 1, ef > 0.0), ev, carry_v)
    fill = jnp.where(ff > 0.0, fv, ev)

    yr_ref[...] = fill * dec

    carry_ref[0] = carry_c_new
    carry_ref[1] = jnp.sum(xb[R - 1:R, 127:128])
    carry_ref[2] = jnp.sum(fill[R - 1:R, 127:128])
    nact_ref[0, 0] = carry_c_new.astype(jnp.int32)


def _main_stage(x, w1r, b1r, w2t, b2r, w3t, b3r):
    return pl.pallas_call(
        _main_body,
        grid=(GRID,),
        in_specs=[
            pl.BlockSpec((BLK, 1), lambda i: (i, 0)),
            pl.BlockSpec((1, HIDDEN), lambda i: (0, 0)),
            pl.BlockSpec((1, HIDDEN), lambda i: (0, 0)),
            pl.BlockSpec((HIDDEN, HIDDEN), lambda i: (0, 0)),
            pl.BlockSpec((1, HIDDEN), lambda i: (0, 0)),
            pl.BlockSpec((HIDDEN, 1), lambda i: (0, 0)),
            pl.BlockSpec((1, 1), lambda i: (0, 0)),
        ],
        out_specs=[
            pl.BlockSpec((R, 128), lambda i: (i, 0)),
            pl.BlockSpec((1, 1), lambda i: (0, 0), memory_space=pltpu.SMEM),
        ],
        out_shape=[
            jax.ShapeDtypeStruct((N // 128, 128), jnp.float32),
            jax.ShapeDtypeStruct((1, 1), jnp.int32),
        ],
        scratch_shapes=[pltpu.SMEM((3,), jnp.float32)],
    )(x, w1r, b1r, w2t, b2r, w3t, b3r)


WB = 16384
WR = WB // 128


def _writer_body(yr_ref, o_ref):
    y = yr_ref[...]
    ii = lax.broadcasted_iota(jnp.int32, (512, 512), 0)
    jj = lax.broadcasted_iota(jnp.int32, (512, 512), 1)
    ident = (ii == jj).astype(jnp.float32)
    yflat = y.reshape(1, WB)
    cols = []
    for k in range(WB // 512):
        yc = yflat[:, k * 512:(k + 1) * 512]
        cols.append(lax.dot_general(
            ident, yc, (((1,), (1,)), ((), ())),
            preferred_element_type=jnp.float32))
    o_ref[...] = jnp.concatenate(cols, axis=0)


def _writer_stage(y_rows):
    return pl.pallas_call(
        _writer_body,
        grid=(N // WB,),
        in_specs=[pl.BlockSpec((WR, 128), lambda i: (i, 0))],
        out_specs=pl.BlockSpec((WB, 1), lambda i: (i, 0)),
        out_shape=jax.ShapeDtypeStruct((N, 1), jnp.float32),
    )(y_rows)


def kernel(x, W1, b1, W2, b2, W3, b3):
    w1r = W1.reshape(1, HIDDEN)
    b1r = b1.reshape(1, HIDDEN)
    w2t = W2.T
    b2r = b2.reshape(1, HIDDEN)
    w3t = W3.reshape(1, HIDDEN).T
    b3r = b3.reshape(1, 1)

    y_rows, nact = _main_stage(x, w1r, b1r, w2t, b2r, w3t, b3r)
    y = y_rows.reshape(N, 1)
    return y, nact[0, 0]
